# baseline (speedup 1.0000x reference)
import functools

import jax
import jax.numpy as jnp
from jax import lax
from jax.experimental import pallas as pl
from jax.experimental.pallas import tpu as pltpu
from jax.experimental.pallas import tpu_sc as plsc


def _sc_segment_sums(NPAD, COLS, NS, CH, C):
    RPT = NPAD // NS
    RCH = RPT // C
    CHQ = CH // 4

    mesh = plsc.VectorSubcoreMesh(core_axis_name="c", subcore_axis_name="s")

    @functools.partial(
        pl.kernel,
        mesh=mesh,
        out_type=jax.ShapeDtypeStruct((2, 2, NPAD, COLS), jnp.float32),
        scratch_types=[
            pltpu.VMEM_SHARED((NPAD, COLS), jnp.float32),
            pltpu.VMEM((CHQ, C), jnp.int32),
            pltpu.VMEM((CHQ, C), jnp.int32),
            pltpu.VMEM((5, C, COLS), jnp.float32),
            pltpu.SemaphoreType.DMA,
            pltpu.SemaphoreType.DMA,
            pltpu.SemaphoreType.DMA,
        ],
    )
    def sc_conv(tab, idx_all, dst_all, out, acc, idx_v, dst_v, buf,
                gsemA, gsemB, ssem):
        c = lax.axis_index("c")
        s = lax.axis_index("s")
        row0 = s * RPT

        def fill_buf_zero():
            def body(i, _):
                r = i // (COLS // 16)
                k = (i % (COLS // 16)) * 16
                buf[0, r, pl.ds(k, 16)] = jnp.zeros((16,), jnp.float32)
                return 0
            lax.fori_loop(0, C * (COLS // 16), body, 0)

        def zero_my_rows():
            def zbody(j, _):
                pltpu.async_copy(buf.at[0], acc.at[pl.ds(row0 + j * C, C)],
                                 ssem)
                return 0
            lax.fori_loop(0, RCH, zbody, 0)

            def zdrain(j, _):
                pltpu.make_async_copy(buf.at[0],
                                      acc.at[pl.ds(row0, C)], ssem).wait()
                return 0
            lax.fori_loop(0, RCH, zdrain, 0)

        fill_buf_zero()
        zero_my_rows()
        plsc.subcore_barrier()

        for conv in range(2):
            for quarter in range(4):
                pltpu.sync_copy(idx_all.at[c, conv, s, quarter], idx_v)
                pltpu.sync_copy(dst_all.at[s, quarter], dst_v)
                pltpu.async_copy(tab.at[idx_v.at[0]], buf.at[0], gsemA)
                pltpu.async_copy(tab.at[idx_v.at[1]], buf.at[1], gsemB)
                pltpu.async_copy(tab.at[idx_v.at[2]], buf.at[2], gsemA)
                pltpu.async_copy(tab.at[idx_v.at[3]], buf.at[3], gsemB)

                def chunk(k, _):
                    slot = k % 5

                    @pl.when(k % 2 == 0)
                    def _wait_even():
                        pltpu.make_async_copy(tab.at[idx_v.at[k]],
                                              buf.at[slot], gsemA).wait()

                    @pl.when(k % 2 == 1)
                    def _wait_odd():
                        pltpu.make_async_copy(tab.at[idx_v.at[k]],
                                              buf.at[slot], gsemB).wait()

                    pltpu.async_copy(buf.at[slot], acc.at[dst_v.at[k]],
                                     ssem, add=True)

                    @pl.when(k >= 1)
                    def _drain():
                        pltpu.make_async_copy(buf.at[slot],
                                              acc.at[dst_v.at[k]], ssem).wait()

                    @pl.when(k < CHQ - 4)
                    def _prefetch():
                        nslot = (k + 4) % 5

                        @pl.when(k % 2 == 0)
                        def _pe():
                            pltpu.async_copy(tab.at[idx_v.at[k + 4]],
                                             buf.at[nslot], gsemA)

                        @pl.when(k % 2 == 1)
                        def _po():
                            pltpu.async_copy(tab.at[idx_v.at[k + 4]],
                                             buf.at[nslot], gsemB)
                    return 0
                lax.fori_loop(0, CHQ, chunk, 0)
                pltpu.make_async_copy(buf.at[0], acc.at[dst_v.at[0]],
                                      ssem).wait()
            plsc.subcore_barrier()

            def wbody(j, _):
                pltpu.async_copy(acc.at[pl.ds(row0 + j * C, C)],
                                 out.at[conv, c, pl.ds(row0 + j * C, C)],
                                 ssem)
                return 0
            lax.fori_loop(0, RCH, wbody, 0)

            def wdrain(j, _):
                pltpu.make_async_copy(acc.at[pl.ds(row0, C)],
                                      out.at[conv, c, pl.ds(row0, C)],
                                      ssem).wait()
                return 0
            lax.fori_loop(0, RCH, wdrain, 0)

            if conv == 0:
                fill_buf_zero()
                zero_my_rows()
                plsc.subcore_barrier()

    return sc_conv


def _tc_fused_kernel(a1lo, a1hi, a2lo, a2hi, w, b, msk, wbil, prompt, bbil,
                     sb1, sb2, o1, o2, h1_s, h2_s, ws_s, ms_s):
    p = pl.program_id(0)
    i = pl.program_id(1)
    BN = a1lo.shape[0]
    D = w.shape[0]
    G = ws_s.shape[0]
    f32 = jnp.float32

    @pl.when(p == 0)
    def _conv():
        a1 = jnp.concatenate([a1lo[...], a1hi[...]], axis=1)
        a2 = jnp.concatenate([a2lo[...], a2hi[...]], axis=1)
        h1v = jnp.maximum(jnp.dot(a1, w[...], preferred_element_type=f32) + b[...], 0.0)
        h2v = jnp.maximum(jnp.dot(a2, w[...], preferred_element_type=f32) + b[...], 0.0)
        h1_s[pl.ds(i * BN, BN), :] = h1v
        h2_s[pl.ds(i * BN, BN), :] = h2v
        m = msk[...]
        ws_s[pl.ds(i, 1), :] = jnp.sum(h1v * m, axis=0)[None, :]
        ms_s[i] = jnp.sum(m)

    @pl.when(p == 1)
    def _score():
        wsum = jnp.sum(ws_s[...], axis=0)
        msum = lax.fori_loop(0, G, lambda g, a: a + ms_s[g], 0.0)
        cvec = jax.nn.sigmoid(wsum / msum).reshape(1, D)
        u = lax.dot_general(cvec, wbil[...], (((1,), (1,)), ((), ())),
                            preferred_element_type=f32)
        u = u * prompt[...]
        h1b = h1_s[pl.ds(i * BN, BN), :]
        h2b = h2_s[pl.ds(i * BN, BN), :]
        s1 = lax.dot_general(u, h1b, (((1,), (1,)), ((), ())),
                             preferred_element_type=f32)
        s2 = lax.dot_general(u, h2b, (((1,), (1,)), ((), ())),
                             preferred_element_type=f32)
        o1[...] = s1[None] + bbil[...][None] + sb1[...]
        o2[...] = s2[None] + bbil[...][None] + sb2[...]


def kernel(x, x_neg, edge_index, seq1, seq2, msk, samp_bias1, samp_bias2,
           W_conv, b_conv, prompt, W_bil, b_bil):
    N, D = x.shape
    E = edge_index.shape[1]
    COLS = D // 2
    NS = 16
    C = 32
    BN = 512
    NPAD = -(-N // (BN * 4)) * (BN * 4)
    G = NPAD // BN
    ES = -(-E // (NS * 4 * C)) * (4 * C)
    CH = ES // C
    EPAD = ES * NS

    f32 = jnp.float32
    tab = jnp.concatenate(
        [x[:, :COLS], x_neg[:, :COLS], x[:, COLS:], x_neg[:, COLS:]], axis=0)

    src = edge_index[0].astype(jnp.int32)
    dst = edge_index[1].astype(jnp.int32)
    src_p = jnp.concatenate([src, jnp.zeros((EPAD - E,), jnp.int32)])
    dst_p = jnp.concatenate([dst, jnp.full((EPAD - E,), N, jnp.int32)])
    block = jnp.arange(2, dtype=jnp.int32)[:, None] * 2 + jnp.arange(2, dtype=jnp.int32)[None, :]
    idx_all = (src_p[None, None, :] + block[:, :, None] * N).reshape(
        2, 2, NS, 4, CH // 4, C)
    dst_all = dst_p.reshape(NS, 4, CH // 4, C)

    agg = _sc_segment_sums(NPAD, COLS, NS, CH, C)(tab, idx_all, dst_all)
    a1lo, a1hi = agg[0, 0], agg[0, 1]
    a2lo, a2hi = agg[1, 0], agg[1, 1]

    msk_p = jnp.pad(msk.astype(f32), ((0, NPAD - N), (0, 0)))
    sb1 = jnp.pad(samp_bias1.astype(f32), (0, NPAD - N)).reshape(G, 1, BN)
    sb2 = jnp.pad(samp_bias2.astype(f32), (0, NPAD - N)).reshape(G, 1, BN)
    o1, o2 = pl.pallas_call(
        _tc_fused_kernel,
        grid=(2, G),
        in_specs=[
            pl.BlockSpec((BN, COLS), lambda p, i: (i * (1 - p), 0)),
            pl.BlockSpec((BN, COLS), lambda p, i: (i * (1 - p), 0)),
            pl.BlockSpec((BN, COLS), lambda p, i: (i * (1 - p), 0)),
            pl.BlockSpec((BN, COLS), lambda p, i: (i * (1 - p), 0)),
            pl.BlockSpec((D, D), lambda p, i: (0, 0)),
            pl.BlockSpec((1, D), lambda p, i: (0, 0)),
            pl.BlockSpec((BN, 1), lambda p, i: (i * (1 - p), 0)),
            pl.BlockSpec((D, D), lambda p, i: (0, 0)),
            pl.BlockSpec((1, D), lambda p, i: (0, 0)),
            pl.BlockSpec((1, 1), lambda p, i: (0, 0)),
            pl.BlockSpec((1, 1, BN), lambda p, i: (i * p, 0, 0)),
            pl.BlockSpec((1, 1, BN), lambda p, i: (i * p, 0, 0)),
        ],
        out_specs=[
            pl.BlockSpec((1, 1, BN), lambda p, i: (i * p, 0, 0)),
            pl.BlockSpec((1, 1, BN), lambda p, i: (i * p, 0, 0)),
        ],
        out_shape=[
            jax.ShapeDtypeStruct((G, 1, BN), f32),
            jax.ShapeDtypeStruct((G, 1, BN), f32),
        ],
        scratch_shapes=[
            pltpu.VMEM((NPAD, D), f32),
            pltpu.VMEM((NPAD, D), f32),
            pltpu.VMEM((G, D), f32),
            pltpu.SMEM((G,), f32),
        ],
    )(a1lo, a1hi, a2lo, a2hi, W_conv, b_conv.reshape(1, D), msk_p,
      W_bil, prompt.reshape(1, D), b_bil.reshape(1, 1), sb1, sb2)

    sc_1 = o1.reshape(NPAD)[:N]
    sc_2 = o2.reshape(NPAD)[:N]
    return jnp.concatenate([sc_1, sc_2], axis=0)


# speedup vs baseline: 1.1279x; 1.1279x over previous
import functools

import jax
import jax.numpy as jnp
from jax import lax
from jax.experimental import pallas as pl
from jax.experimental.pallas import tpu as pltpu
from jax.experimental.pallas import tpu_sc as plsc


def _sc_segment_sums(NPAD, COLS, NS, CH, C):
    RPT = NPAD // NS
    RCH = RPT // C
    CHQ = CH // 4

    mesh = plsc.VectorSubcoreMesh(core_axis_name="c", subcore_axis_name="s")

    @functools.partial(
        pl.kernel,
        mesh=mesh,
        out_type=jax.ShapeDtypeStruct((2, 2, NPAD, COLS), jnp.float32),
        scratch_types=[
            pltpu.VMEM_SHARED((NPAD, COLS), jnp.float32),
            pltpu.VMEM((CHQ, C), jnp.int32),
            pltpu.VMEM((CHQ, C), jnp.int32),
            pltpu.VMEM((5, C, COLS), jnp.float32),
            pltpu.SemaphoreType.DMA,
            pltpu.SemaphoreType.DMA,
            pltpu.SemaphoreType.DMA,
        ],
    )
    def sc_conv(xa, xb, idx_all, dst_all, out, acc, idx_v, dst_v, buf,
                gsemA, gsemB, ssem):
        c = lax.axis_index("c")
        s = lax.axis_index("s")
        row0 = s * RPT

        def fill_buf_zero():
            def body(i, _):
                r = i // (COLS // 16)
                k = (i % (COLS // 16)) * 16
                buf[0, r, pl.ds(k, 16)] = jnp.zeros((16,), jnp.float32)
                return 0
            lax.fori_loop(0, C * (COLS // 16), body, 0)

        def zero_my_rows():
            def zbody(j, _):
                pltpu.async_copy(buf.at[0], acc.at[pl.ds(row0 + j * C, C)],
                                 ssem)
                return 0
            lax.fori_loop(0, RCH, zbody, 0)

            def zdrain(j, _):
                pltpu.make_async_copy(buf.at[0],
                                      acc.at[pl.ds(row0, C)], ssem).wait()
                return 0
            lax.fori_loop(0, RCH, zdrain, 0)

        fill_buf_zero()
        zero_my_rows()
        plsc.subcore_barrier()

        for conv in range(2):
            xsrc = (xa, xb)[conv]
            for half in range(2):
                lo = half * COLS

                @pl.when(c == half)
                def _run():
                    def gather(k, slot, sem):
                        pltpu.async_copy(
                            xsrc.at[idx_v.at[k], pl.ds(lo, COLS)],
                            buf.at[slot], sem)

                    def gwait(k, slot, sem):
                        pltpu.make_async_copy(
                            xsrc.at[idx_v.at[k], pl.ds(lo, COLS)],
                            buf.at[slot], sem).wait()

                    for quarter in range(4):
                        pltpu.sync_copy(idx_all.at[s, quarter], idx_v)
                        pltpu.sync_copy(dst_all.at[s, quarter], dst_v)
                        gather(0, 0, gsemA)
                        gather(1, 1, gsemB)
                        gather(2, 2, gsemA)
                        gather(3, 3, gsemB)

                        def chunk(k, _):
                            slot = k % 5

                            @pl.when(k % 2 == 0)
                            def _wait_even():
                                gwait(k, slot, gsemA)

                            @pl.when(k % 2 == 1)
                            def _wait_odd():
                                gwait(k, slot, gsemB)

                            pltpu.async_copy(buf.at[slot],
                                             acc.at[dst_v.at[k]],
                                             ssem, add=True)

                            @pl.when(k >= 1)
                            def _drain():
                                pltpu.make_async_copy(
                                    buf.at[slot], acc.at[dst_v.at[k]],
                                    ssem).wait()

                            @pl.when(k < CHQ - 4)
                            def _prefetch():
                                nslot = (k + 4) % 5

                                @pl.when(k % 2 == 0)
                                def _pe():
                                    gather(k + 4, nslot, gsemA)

                                @pl.when(k % 2 == 1)
                                def _po():
                                    gather(k + 4, nslot, gsemB)
                            return 0
                        lax.fori_loop(0, CHQ, chunk, 0)
                        pltpu.make_async_copy(buf.at[0], acc.at[dst_v.at[0]],
                                              ssem).wait()
            plsc.subcore_barrier()

            def wbody(j, _):
                pltpu.async_copy(acc.at[pl.ds(row0 + j * C, C)],
                                 out.at[conv, c, pl.ds(row0 + j * C, C)],
                                 ssem)
                return 0
            lax.fori_loop(0, RCH, wbody, 0)

            def wdrain(j, _):
                pltpu.make_async_copy(acc.at[pl.ds(row0, C)],
                                      out.at[conv, c, pl.ds(row0, C)],
                                      ssem).wait()
                return 0
            lax.fori_loop(0, RCH, wdrain, 0)

            if conv == 0:
                fill_buf_zero()
                zero_my_rows()
                plsc.subcore_barrier()

    return sc_conv


def _tc_fused_kernel(a1lo, a1hi, a2lo, a2hi, w, b, msk, wbil, prompt, bbil,
                     sb1, sb2, o1, o2, h1_s, h2_s, ws_s, ms_s):
    p = pl.program_id(0)
    i = pl.program_id(1)
    BN = a1lo.shape[0]
    D = w.shape[0]
    G = ws_s.shape[0]
    f32 = jnp.float32

    @pl.when(p == 0)
    def _conv():
        a1 = jnp.concatenate([a1lo[...], a1hi[...]], axis=1)
        a2 = jnp.concatenate([a2lo[...], a2hi[...]], axis=1)
        h1v = jnp.maximum(jnp.dot(a1, w[...], preferred_element_type=f32) + b[...], 0.0)
        h2v = jnp.maximum(jnp.dot(a2, w[...], preferred_element_type=f32) + b[...], 0.0)
        h1_s[pl.ds(i * BN, BN), :] = h1v
        h2_s[pl.ds(i * BN, BN), :] = h2v
        m = msk[...]
        ws_s[pl.ds(i, 1), :] = jnp.sum(h1v * m, axis=0)[None, :]
        ms_s[i] = jnp.sum(m)

    @pl.when(p == 1)
    def _score():
        wsum = jnp.sum(ws_s[...], axis=0)
        msum = lax.fori_loop(0, G, lambda g, a: a + ms_s[g], 0.0)
        cvec = jax.nn.sigmoid(wsum / msum).reshape(1, D)
        u = lax.dot_general(cvec, wbil[...], (((1,), (1,)), ((), ())),
                            preferred_element_type=f32)
        u = u * prompt[...]
        h1b = h1_s[pl.ds(i * BN, BN), :]
        h2b = h2_s[pl.ds(i * BN, BN), :]
        s1 = lax.dot_general(u, h1b, (((1,), (1,)), ((), ())),
                             preferred_element_type=f32)
        s2 = lax.dot_general(u, h2b, (((1,), (1,)), ((), ())),
                             preferred_element_type=f32)
        o1[...] = s1[None] + bbil[...][None] + sb1[...]
        o2[...] = s2[None] + bbil[...][None] + sb2[...]


def kernel(x, x_neg, edge_index, seq1, seq2, msk, samp_bias1, samp_bias2,
           W_conv, b_conv, prompt, W_bil, b_bil):
    N, D = x.shape
    E = edge_index.shape[1]
    COLS = D // 2
    NS = 16
    C = 32
    BN = 512
    NPAD = -(-N // (BN * 4)) * (BN * 4)
    G = NPAD // BN
    ES = -(-E // (NS * 4 * C)) * (4 * C)
    CH = ES // C
    EPAD = ES * NS

    f32 = jnp.float32
    src = edge_index[0].astype(jnp.int32)
    dst = edge_index[1].astype(jnp.int32)
    src_p = jnp.concatenate([src, jnp.zeros((EPAD - E,), jnp.int32)])
    dst_p = jnp.concatenate([dst, jnp.full((EPAD - E,), N, jnp.int32)])
    idx_all = src_p.reshape(NS, 4, CH // 4, C)
    dst_all = dst_p.reshape(NS, 4, CH // 4, C)

    agg = _sc_segment_sums(NPAD, COLS, NS, CH, C)(x, x_neg, idx_all, dst_all)
    a1lo, a1hi = agg[0, 0], agg[0, 1]
    a2lo, a2hi = agg[1, 0], agg[1, 1]

    msk_p = jnp.pad(msk.astype(f32), ((0, NPAD - N), (0, 0)))
    sb1 = jnp.pad(samp_bias1.astype(f32), (0, NPAD - N)).reshape(G, 1, BN)
    sb2 = jnp.pad(samp_bias2.astype(f32), (0, NPAD - N)).reshape(G, 1, BN)
    o1, o2 = pl.pallas_call(
        _tc_fused_kernel,
        grid=(2, G),
        in_specs=[
            pl.BlockSpec((BN, COLS), lambda p, i: (i * (1 - p), 0)),
            pl.BlockSpec((BN, COLS), lambda p, i: (i * (1 - p), 0)),
            pl.BlockSpec((BN, COLS), lambda p, i: (i * (1 - p), 0)),
            pl.BlockSpec((BN, COLS), lambda p, i: (i * (1 - p), 0)),
            pl.BlockSpec((D, D), lambda p, i: (0, 0)),
            pl.BlockSpec((1, D), lambda p, i: (0, 0)),
            pl.BlockSpec((BN, 1), lambda p, i: (i * (1 - p), 0)),
            pl.BlockSpec((D, D), lambda p, i: (0, 0)),
            pl.BlockSpec((1, D), lambda p, i: (0, 0)),
            pl.BlockSpec((1, 1), lambda p, i: (0, 0)),
            pl.BlockSpec((1, 1, BN), lambda p, i: (i * p, 0, 0)),
            pl.BlockSpec((1, 1, BN), lambda p, i: (i * p, 0, 0)),
        ],
        out_specs=[
            pl.BlockSpec((1, 1, BN), lambda p, i: (i * p, 0, 0)),
            pl.BlockSpec((1, 1, BN), lambda p, i: (i * p, 0, 0)),
        ],
        out_shape=[
            jax.ShapeDtypeStruct((G, 1, BN), f32),
            jax.ShapeDtypeStruct((G, 1, BN), f32),
        ],
        scratch_shapes=[
            pltpu.VMEM((NPAD, D), f32),
            pltpu.VMEM((NPAD, D), f32),
            pltpu.VMEM((G, D), f32),
            pltpu.SMEM((G,), f32),
        ],
    )(a1lo, a1hi, a2lo, a2hi, W_conv, b_conv.reshape(1, D), msk_p,
      W_bil, prompt.reshape(1, D), b_bil.reshape(1, 1), sb1, sb2)

    sc_1 = o1.reshape(NPAD)[:N]
    sc_2 = o2.reshape(NPAD)[:N]
    return jnp.concatenate([sc_1, sc_2], axis=0)


# speedup vs baseline: 1.2950x; 1.1481x over previous
import functools

import jax
import jax.numpy as jnp
from jax import lax
from jax.experimental import pallas as pl
from jax.experimental.pallas import tpu as pltpu
from jax.experimental.pallas import tpu_sc as plsc


def _sc_segment_sums(NPAD, NACC, COLS, NS, CH, C):
    RPT = NACC // NS
    RC = 8
    RCH = RPT // RC
    CHQ = CH // 4

    mesh = plsc.VectorSubcoreMesh(core_axis_name="c", subcore_axis_name="s")

    @functools.partial(
        pl.kernel,
        mesh=mesh,
        out_type=jax.ShapeDtypeStruct((2, 2, NPAD, COLS), jnp.float32),
        scratch_types=[
            pltpu.VMEM_SHARED((NACC, COLS), jnp.float32),
            pltpu.VMEM((CHQ, C), jnp.int32),
            pltpu.VMEM((CHQ, C), jnp.int32),
            pltpu.VMEM((7, C, COLS), jnp.float32),
            pltpu.SemaphoreType.DMA,
            pltpu.SemaphoreType.DMA,
            pltpu.SemaphoreType.DMA,
            pltpu.SemaphoreType.DMA,
        ],
    )
    def sc_conv(xa, xb, idx_all, dst_all, out, acc, idx_v, dst_v, buf,
                gsemA, gsemB, gsemC, ssem):
        c = lax.axis_index("c")
        s = lax.axis_index("s")
        row0 = s * RPT

        def fill_buf_zero():
            def body(i, _):
                r = i // (COLS // 16)
                k = (i % (COLS // 16)) * 16
                buf[0, r, pl.ds(k, 16)] = jnp.zeros((16,), jnp.float32)
                return 0
            lax.fori_loop(0, C * (COLS // 16), body, 0)

        def zero_my_rows():
            def zbody(j, _):
                pltpu.async_copy(buf.at[0, pl.ds(0, RC)],
                                 acc.at[pl.ds(row0 + j * RC, RC)], ssem)
                return 0
            lax.fori_loop(0, RCH, zbody, 0)

            def zdrain(j, _):
                pltpu.make_async_copy(buf.at[0, pl.ds(0, RC)],
                                      acc.at[pl.ds(row0, RC)], ssem).wait()
                return 0
            lax.fori_loop(0, RCH, zdrain, 0)

        fill_buf_zero()
        zero_my_rows()
        plsc.subcore_barrier()

        lo = pl.multiple_of(c * COLS, COLS)

        for conv in range(2):
            xsrc = (xa, xb)[conv]

            def gather(k, slot, sem):
                pltpu.async_copy(
                    xsrc.at[idx_v.at[k], pl.ds(lo, COLS)],
                    buf.at[slot], sem)

            def gwait(k, slot, sem):
                pltpu.make_async_copy(
                    xsrc.at[idx_v.at[k], pl.ds(lo, COLS)],
                    buf.at[slot], sem).wait()

            def qloop(q, _):
                pltpu.sync_copy(idx_all.at[s, q], idx_v)
                pltpu.sync_copy(dst_all.at[s, q], dst_v)
                gather(0, 0, gsemA)
                gather(1, 1, gsemB)
                gather(2, 2, gsemC)
                gather(3, 3, gsemA)
                gather(4, 4, gsemB)
                gather(5, 5, gsemC)

                def chunk(k, _):
                    slot = k % 7

                    @pl.when(k % 3 == 0)
                    def _wait_a():
                        gwait(k, slot, gsemA)

                    @pl.when(k % 3 == 1)
                    def _wait_b():
                        gwait(k, slot, gsemB)

                    @pl.when(k % 3 == 2)
                    def _wait_c():
                        gwait(k, slot, gsemC)

                    pltpu.async_copy(buf.at[slot], acc.at[dst_v.at[k]],
                                     ssem, add=True)

                    @pl.when(k >= 1)
                    def _drain():
                        pltpu.make_async_copy(
                            buf.at[slot], acc.at[dst_v.at[k]], ssem).wait()

                    @pl.when(k < CHQ - 6)
                    def _prefetch():
                        nslot = (k + 6) % 7

                        @pl.when(k % 3 == 0)
                        def _pa():
                            gather(k + 6, nslot, gsemA)

                        @pl.when(k % 3 == 1)
                        def _pb():
                            gather(k + 6, nslot, gsemB)

                        @pl.when(k % 3 == 2)
                        def _pc():
                            gather(k + 6, nslot, gsemC)
                    return 0
                lax.fori_loop(0, CHQ, chunk, 0)
                pltpu.make_async_copy(buf.at[0], acc.at[dst_v.at[0]],
                                      ssem).wait()
                return 0
            lax.fori_loop(0, 4, qloop, 0)
            plsc.subcore_barrier()

            def wbody(j, _):
                pltpu.async_copy(acc.at[pl.ds(row0 + j * RC, RC)],
                                 out.at[conv, c, pl.ds(row0 + j * RC, RC)],
                                 ssem)
                return 0
            lax.fori_loop(0, RCH, wbody, 0)

            def wdrain(j, _):
                pltpu.make_async_copy(acc.at[pl.ds(row0, RC)],
                                      out.at[conv, c, pl.ds(row0, RC)],
                                      ssem).wait()
                return 0
            lax.fori_loop(0, RCH, wdrain, 0)

            if conv == 0:
                fill_buf_zero()
                zero_my_rows()
                plsc.subcore_barrier()

    return sc_conv


def _tc_fused_kernel(a1lo, a1hi, a2lo, a2hi, w, b, msk, wbil, prompt, bbil,
                     sb1, sb2, o1, o2, h1_s, h2_s, ws_s, ms_s):
    p = pl.program_id(0)
    i = pl.program_id(1)
    BN = a1lo.shape[0]
    D = w.shape[0]
    G = ws_s.shape[0]
    f32 = jnp.float32

    @pl.when(p == 0)
    def _conv():
        a1 = jnp.concatenate([a1lo[...], a1hi[...]], axis=1)
        a2 = jnp.concatenate([a2lo[...], a2hi[...]], axis=1)
        h1v = jnp.maximum(jnp.dot(a1, w[...], preferred_element_type=f32) + b[...], 0.0)
        h2v = jnp.maximum(jnp.dot(a2, w[...], preferred_element_type=f32) + b[...], 0.0)
        h1_s[pl.ds(i * BN, BN), :] = h1v
        h2_s[pl.ds(i * BN, BN), :] = h2v
        m = msk[...]
        hm = jnp.where(m > 0.0, h1v * m, 0.0)
        ws_s[pl.ds(i, 1), :] = jnp.sum(hm, axis=0)[None, :]
        ms_s[i] = jnp.sum(m)

    @pl.when(p == 1)
    def _score():
        wsum = jnp.sum(ws_s[...], axis=0)
        msum = lax.fori_loop(0, G, lambda g, a: a + ms_s[g], 0.0)
        cvec = jax.nn.sigmoid(wsum / msum).reshape(1, D)
        u = lax.dot_general(cvec, wbil[...], (((1,), (1,)), ((), ())),
                            preferred_element_type=f32)
        u = u * prompt[...]
        h1b = h1_s[pl.ds(i * BN, BN), :]
        h2b = h2_s[pl.ds(i * BN, BN), :]
        s1 = lax.dot_general(u, h1b, (((1,), (1,)), ((), ())),
                             preferred_element_type=f32)
        s2 = lax.dot_general(u, h2b, (((1,), (1,)), ((), ())),
                             preferred_element_type=f32)
        o1[...] = s1[None] + bbil[...][None] + sb1[...]
        o2[...] = s2[None] + bbil[...][None] + sb2[...]


def kernel(x, x_neg, edge_index, seq1, seq2, msk, samp_bias1, samp_bias2,
           W_conv, b_conv, prompt, W_bil, b_bil):
    N, D = x.shape
    E = edge_index.shape[1]
    COLS = D // 2
    NS = 16
    C = 24
    BN = 512
    NPAD = -(-N // (BN * 4)) * (BN * 4)
    G = NPAD // BN
    ES = -(-E // (NS * 4 * C)) * (4 * C)
    CH = ES // C
    EPAD = ES * NS

    f32 = jnp.float32
    src = edge_index[0].astype(jnp.int32)
    dst = edge_index[1].astype(jnp.int32)
    src_p = jnp.concatenate([src, jnp.zeros((EPAD - E,), jnp.int32)])
    dst_p = jnp.concatenate([dst, jnp.full((EPAD - E,), N, jnp.int32)])
    idx_all = src_p.reshape(NS, 4, CH // 4, C)
    dst_all = dst_p.reshape(NS, 4, CH // 4, C)

    NACC = -(-(N + 1) // (NS * 8)) * (NS * 8)
    agg = _sc_segment_sums(NPAD, NACC, COLS, NS, CH, C)(x, x_neg, idx_all, dst_all)
    a1lo, a1hi = agg[0, 0], agg[0, 1]
    a2lo, a2hi = agg[1, 0], agg[1, 1]

    msk_p = jnp.pad(msk.astype(f32), ((0, NPAD - N), (0, 0)))
    sb1 = jnp.pad(samp_bias1.astype(f32), (0, NPAD - N)).reshape(G, 1, BN)
    sb2 = jnp.pad(samp_bias2.astype(f32), (0, NPAD - N)).reshape(G, 1, BN)
    o1, o2 = pl.pallas_call(
        _tc_fused_kernel,
        grid=(2, G),
        in_specs=[
            pl.BlockSpec((BN, COLS), lambda p, i: (i * (1 - p), 0)),
            pl.BlockSpec((BN, COLS), lambda p, i: (i * (1 - p), 0)),
            pl.BlockSpec((BN, COLS), lambda p, i: (i * (1 - p), 0)),
            pl.BlockSpec((BN, COLS), lambda p, i: (i * (1 - p), 0)),
            pl.BlockSpec((D, D), lambda p, i: (0, 0)),
            pl.BlockSpec((1, D), lambda p, i: (0, 0)),
            pl.BlockSpec((BN, 1), lambda p, i: (i * (1 - p), 0)),
            pl.BlockSpec((D, D), lambda p, i: (0, 0)),
            pl.BlockSpec((1, D), lambda p, i: (0, 0)),
            pl.BlockSpec((1, 1), lambda p, i: (0, 0)),
            pl.BlockSpec((1, 1, BN), lambda p, i: (i * p, 0, 0)),
            pl.BlockSpec((1, 1, BN), lambda p, i: (i * p, 0, 0)),
        ],
        out_specs=[
            pl.BlockSpec((1, 1, BN), lambda p, i: (i * p, 0, 0)),
            pl.BlockSpec((1, 1, BN), lambda p, i: (i * p, 0, 0)),
        ],
        out_shape=[
            jax.ShapeDtypeStruct((G, 1, BN), f32),
            jax.ShapeDtypeStruct((G, 1, BN), f32),
        ],
        scratch_shapes=[
            pltpu.VMEM((NPAD, D), f32),
            pltpu.VMEM((NPAD, D), f32),
            pltpu.VMEM((G, D), f32),
            pltpu.SMEM((G,), f32),
        ],
    )(a1lo, a1hi, a2lo, a2hi, W_conv, b_conv.reshape(1, D), msk_p,
      W_bil, prompt.reshape(1, D), b_bil.reshape(1, 1), sb1, sb2)

    sc_1 = o1.reshape(NPAD)[:N]
    sc_2 = o2.reshape(NPAD)[:N]
    return jnp.concatenate([sc_1, sc_2], axis=0)
